# Initial kernel scaffold; baseline (speedup 1.0000x reference)
import functools

import jax
import jax.numpy as jnp
from jax import lax
from jax.experimental import pallas as pl
from jax.experimental.pallas import tpu as pltpu
from jax.experimental.pallas import tpu_sc as plsc

N = 10000
E = 320000
D = 128
NC, NS, L = 2, 16, 16
NW = NC * NS
NPAD = 10240
EPW = E // NW
CH = 80
NCHUNK = EPW // CH
RPT = NPAD // NS

_MESH = plsc.VectorSubcoreMesh(core_axis_name="c", subcore_axis_name="s")


@functools.partial(
    pl.kernel,
    mesh=_MESH,
    out_type=jax.ShapeDtypeStruct((NC, NPAD, L), jnp.float32),
    scratch_types=[
        pltpu.VMEM((CH,), jnp.int32),
        pltpu.VMEM((CH, L), jnp.float32),
        pltpu.VMEM_SHARED((NPAD, L), jnp.float32),
    ],
)
def _deg_kernel(dst_hbm, ones_hbm, zeros_hbm, out_hbm, didx_v, ones_v, deg_sh):
    c = lax.axis_index("c")
    s = lax.axis_index("s")
    wid = s * NC + c
    pltpu.sync_copy(ones_hbm, ones_v)
    pltpu.sync_copy(zeros_hbm.at[pl.ds(s * RPT, RPT)],
                    deg_sh.at[pl.ds(s * RPT, RPT)])
    plsc.subcore_barrier()

    def body(i, carry):
        base = wid * EPW + i * CH
        pltpu.sync_copy(dst_hbm.at[pl.ds(base, CH)], didx_v)
        pltpu.sync_copy(ones_v, deg_sh.at[didx_v], add=True)
        return carry

    lax.fori_loop(0, NCHUNK, body, 0)
    plsc.subcore_barrier()
    pltpu.sync_copy(deg_sh.at[pl.ds(s * RPT, RPT)],
                    out_hbm.at[c, pl.ds(s * RPT, RPT)])


@functools.partial(
    pl.kernel,
    mesh=_MESH,
    out_type=jax.ShapeDtypeStruct((NC, NPAD, D), jnp.float32),
    scratch_types=[
        pltpu.VMEM((CH,), jnp.int32),
        pltpu.VMEM((CH,), jnp.int32),
        pltpu.VMEM((CH, D), jnp.float32),
        pltpu.VMEM_SHARED((NPAD, D), jnp.float32),
        pltpu.SemaphoreType.DMA,
    ],
)
def _scatter_kernel(hs_hbm, src_hbm, dst_hbm, out_hbm,
                    sidx_v, didx_v, rows_v, acc_sh, sem):
    c = lax.axis_index("c")
    s = lax.axis_index("s")
    wid = s * NC + c
    pltpu.sync_copy(hs_hbm.at[pl.ds(s * RPT, RPT)],
                    acc_sh.at[pl.ds(s * RPT, RPT)])
    plsc.subcore_barrier()

    def body(i, carry):
        base = wid * EPW + i * CH
        pltpu.sync_copy(src_hbm.at[pl.ds(base, CH)], sidx_v)
        pltpu.sync_copy(dst_hbm.at[pl.ds(base, CH)], didx_v)
        pltpu.async_copy(hs_hbm.at[sidx_v], rows_v, sem).wait()
        pltpu.sync_copy(rows_v, acc_sh.at[didx_v], add=True)
        return carry

    lax.fori_loop(0, NCHUNK, body, 0)
    plsc.subcore_barrier()
    pltpu.sync_copy(acc_sh.at[pl.ds(s * RPT, RPT)],
                    out_hbm.at[c, pl.ds(s * RPT, RPT)])


BLK = 1280


def _l1_body(x_ref, dp_ref, w_ref, hs_ref, g_ref):
    p = dp_ref[...]
    deg = 1.0 + p[0, :, 0:1] + p[1, :, 0:1]
    g = lax.rsqrt(deg)
    h = jnp.dot(x_ref[...], w_ref[...], preferred_element_type=jnp.float32)
    hs_ref[...] = h * g
    g_ref[...] = g


def _l2_body(a0_ref, a1_ref, hs1_ref, g_ref, b_ref, w_ref, hs2_ref):
    g = g_ref[...]
    y = g * (a0_ref[...] + a1_ref[...] - hs1_ref[...]) + b_ref[...]
    y = jnp.maximum(y, 0.0)
    hs2_ref[...] = jnp.dot(y, w_ref[...],
                           preferred_element_type=jnp.float32) * g


def _l3_body(a0_ref, a1_ref, hs2_ref, g_ref, b_ref, out_ref):
    g = g_ref[...]
    out_ref[...] = g * (a0_ref[...] + a1_ref[...] - hs2_ref[...]) + b_ref[...]


_row = pl.BlockSpec((BLK, D), lambda i: (i, 0))
_gcol = pl.BlockSpec((BLK, 1), lambda i: (i, 0))
_full_w = pl.BlockSpec((D, D), lambda i: (0, 0))
_bias = pl.BlockSpec((1, D), lambda i: (0, 0))
_GRID = (NPAD // BLK,)


def _l1_call(x_pad, degp, W1):
    return pl.pallas_call(
        _l1_body,
        grid=_GRID,
        in_specs=[_row,
                  pl.BlockSpec((NC, BLK, L), lambda i: (0, i, 0)),
                  _full_w],
        out_specs=[_row, _gcol],
        out_shape=[jax.ShapeDtypeStruct((NPAD, D), jnp.float32),
                   jax.ShapeDtypeStruct((NPAD, 1), jnp.float32)],
    )(x_pad, degp, W1)


def _l2_call(a0, a1, hs1, g, b1, W2):
    return pl.pallas_call(
        _l2_body,
        grid=_GRID,
        in_specs=[_row, _row, _row, _gcol, _bias, _full_w],
        out_specs=_row,
        out_shape=jax.ShapeDtypeStruct((NPAD, D), jnp.float32),
    )(a0, a1, hs1, g, b1, W2)


def _l3_call(a0, a1, hs2, g, b2):
    return pl.pallas_call(
        _l3_body,
        grid=_GRID,
        in_specs=[_row, _row, _row, _gcol, _bias],
        out_specs=_row,
        out_shape=jax.ShapeDtypeStruct((NPAD, D), jnp.float32),
    )(a0, a1, hs2, g, b2)


def kernel(x, edge_index, W1, b1, W2, b2):
    src = edge_index[0].astype(jnp.int32)
    dst = edge_index[1].astype(jnp.int32)
    x_pad = jnp.zeros((NPAD, D), jnp.float32).at[:N].set(x)
    ones_ch = jnp.ones((CH, L), jnp.float32)
    zeros_pad = jnp.zeros((NPAD, L), jnp.float32)

    degp = _deg_kernel(dst, ones_ch, zeros_pad)
    hs1, g = _l1_call(x_pad, degp, W1)
    acc1 = _scatter_kernel(hs1, src, dst)
    hs2 = _l2_call(acc1[0], acc1[1], hs1, g,
                   b1.reshape(1, D).astype(jnp.float32), W2)
    acc2 = _scatter_kernel(hs2, src, dst)
    out = _l3_call(acc2[0], acc2[1], hs2, g,
                   b2.reshape(1, D).astype(jnp.float32))
    return out[:N]


# speedup vs baseline: 13.9131x; 13.9131x over previous
import functools

import jax
import jax.numpy as jnp
from jax import lax
from jax.experimental import pallas as pl
from jax.experimental.pallas import tpu as pltpu
from jax.experimental.pallas import tpu_sc as plsc

N = 10000
E = 320000
D = 128
NC, NS, L = 2, 16, 16
NW = NC * NS
NPAD = 10240
EPW = E // NW
CH = 80
NCHUNK = EPW // CH
RPT = NPAD // NS

_MESH = plsc.VectorSubcoreMesh(core_axis_name="c", subcore_axis_name="s")


@functools.partial(
    pl.kernel,
    mesh=_MESH,
    compiler_params=pltpu.CompilerParams(needs_layout_passes=False),
    out_type=jax.ShapeDtypeStruct((NW, NPAD), jnp.float32),
    scratch_types=[
        pltpu.VMEM((EPW,), jnp.int32),
        pltpu.VMEM((NPAD,), jnp.float32),
    ],
)
def _deg_kernel(dst_hbm, out_hbm, didx_v, hist_v):
    c = lax.axis_index("c")
    s = lax.axis_index("s")
    wid = s * NC + c
    pltpu.sync_copy(dst_hbm.at[pl.ds(wid * EPW, EPW)], didx_v)

    def zbody(i, carry):
        hist_v[pl.ds(i * L, L)] = jnp.zeros((L,), jnp.float32)
        return carry

    lax.fori_loop(0, NPAD // L, zbody, 0)

    ones = jnp.ones((L,), jnp.float32)

    def body(i, carry):
        idx = didx_v[pl.ds(i * L, L)]
        plsc.addupdate_scatter(hist_v, [idx], ones)
        return carry

    lax.fori_loop(0, EPW // L, body, 0)
    pltpu.sync_copy(hist_v, out_hbm.at[wid])


@functools.partial(
    pl.kernel,
    mesh=_MESH,
    out_type=jax.ShapeDtypeStruct((NC, NPAD, D), jnp.float32),
    scratch_types=[
        pltpu.VMEM((CH,), jnp.int32),
        pltpu.VMEM((CH,), jnp.int32),
        pltpu.VMEM((CH, D), jnp.float32),
        pltpu.VMEM_SHARED((NPAD, D), jnp.float32),
        pltpu.SemaphoreType.DMA,
    ],
)
def _scatter_kernel(hs_hbm, src_hbm, dst_hbm, out_hbm,
                    sidx_v, didx_v, rows_v, acc_sh, sem):
    c = lax.axis_index("c")
    s = lax.axis_index("s")
    wid = s * NC + c
    pltpu.sync_copy(hs_hbm.at[pl.ds(s * RPT, RPT)],
                    acc_sh.at[pl.ds(s * RPT, RPT)])
    plsc.subcore_barrier()

    def body(i, carry):
        base = wid * EPW + i * CH
        pltpu.sync_copy(src_hbm.at[pl.ds(base, CH)], sidx_v)
        pltpu.sync_copy(dst_hbm.at[pl.ds(base, CH)], didx_v)
        pltpu.async_copy(hs_hbm.at[sidx_v], rows_v, sem).wait()
        pltpu.sync_copy(rows_v, acc_sh.at[didx_v], add=True)
        return carry

    lax.fori_loop(0, NCHUNK, body, 0)
    plsc.subcore_barrier()
    pltpu.sync_copy(acc_sh.at[pl.ds(s * RPT, RPT)],
                    out_hbm.at[c, pl.ds(s * RPT, RPT)])


BLK = 1280


def _degsum_body(p_ref, cnt_ref):
    cnt_ref[...] = jnp.sum(p_ref[...], axis=0, keepdims=True)


def _degsum_call(degp):
    return pl.pallas_call(
        _degsum_body,
        out_shape=jax.ShapeDtypeStruct((1, NPAD), jnp.float32),
    )(degp)


def _l1_body(x_ref, deg_ref, w_ref, hs_ref, g_ref):
    g = lax.rsqrt(1.0 + deg_ref[...])
    h = jnp.dot(x_ref[...], w_ref[...], preferred_element_type=jnp.float32)
    hs_ref[...] = h * g
    g_ref[...] = g


def _l2_body(a0_ref, a1_ref, hs1_ref, g_ref, b_ref, w_ref, hs2_ref):
    g = g_ref[...]
    y = g * (a0_ref[...] + a1_ref[...] - hs1_ref[...]) + b_ref[...]
    y = jnp.maximum(y, 0.0)
    hs2_ref[...] = jnp.dot(y, w_ref[...],
                           preferred_element_type=jnp.float32) * g


def _l3_body(a0_ref, a1_ref, hs2_ref, g_ref, b_ref, out_ref):
    g = g_ref[...]
    out_ref[...] = g * (a0_ref[...] + a1_ref[...] - hs2_ref[...]) + b_ref[...]


_row = pl.BlockSpec((BLK, D), lambda i: (i, 0))
_gcol = pl.BlockSpec((BLK, 1), lambda i: (i, 0))
_full_w = pl.BlockSpec((D, D), lambda i: (0, 0))
_bias = pl.BlockSpec((1, D), lambda i: (0, 0))
_GRID = (NPAD // BLK,)


def _l1_call(x_pad, deg_col, W1):
    return pl.pallas_call(
        _l1_body,
        grid=_GRID,
        in_specs=[_row, _gcol, _full_w],
        out_specs=[_row, _gcol],
        out_shape=[jax.ShapeDtypeStruct((NPAD, D), jnp.float32),
                   jax.ShapeDtypeStruct((NPAD, 1), jnp.float32)],
    )(x_pad, deg_col, W1)


def _l2_call(a0, a1, hs1, g, b1, W2):
    return pl.pallas_call(
        _l2_body,
        grid=_GRID,
        in_specs=[_row, _row, _row, _gcol, _bias, _full_w],
        out_specs=_row,
        out_shape=jax.ShapeDtypeStruct((NPAD, D), jnp.float32),
    )(a0, a1, hs1, g, b1, W2)


def _l3_call(a0, a1, hs2, g, b2):
    return pl.pallas_call(
        _l3_body,
        grid=_GRID,
        in_specs=[_row, _row, _row, _gcol, _bias],
        out_specs=_row,
        out_shape=jax.ShapeDtypeStruct((NPAD, D), jnp.float32),
    )(a0, a1, hs2, g, b2)


def kernel(x, edge_index, W1, b1, W2, b2):
    src = edge_index[0].astype(jnp.int32)
    dst = edge_index[1].astype(jnp.int32)
    x_pad = jnp.zeros((NPAD, D), jnp.float32).at[:N].set(x)

    degp = _deg_kernel(dst)
    deg_col = _degsum_call(degp).reshape(NPAD, 1)
    hs1, g = _l1_call(x_pad, deg_col, W1)
    acc1 = _scatter_kernel(hs1, src, dst)
    hs2 = _l2_call(acc1[0], acc1[1], hs1, g,
                   b1.reshape(1, D).astype(jnp.float32), W2)
    acc2 = _scatter_kernel(hs2, src, dst)
    out = _l3_call(acc2[0], acc2[1], hs2, g,
                   b2.reshape(1, D).astype(jnp.float32))
    return out[:N]


# speedup vs baseline: 28.6673x; 2.0605x over previous
import functools

import jax
import jax.numpy as jnp
from jax import lax
from jax.experimental import pallas as pl
from jax.experimental.pallas import tpu as pltpu
from jax.experimental.pallas import tpu_sc as plsc

N = 10000
E = 320000
D = 128
NC, NS, L = 2, 16, 16
NW = NC * NS
NPAD = 10240
EPW = E // NW
CH = 80
NCHUNK = EPW // CH
RPT = NPAD // NS

_MESH = plsc.VectorSubcoreMesh(core_axis_name="c", subcore_axis_name="s")


@functools.partial(
    pl.kernel,
    mesh=_MESH,
    compiler_params=pltpu.CompilerParams(needs_layout_passes=False),
    out_type=jax.ShapeDtypeStruct((NW, NPAD), jnp.float32),
    scratch_types=[
        pltpu.VMEM((EPW,), jnp.int32),
        pltpu.VMEM((NPAD,), jnp.float32),
    ],
)
def _deg_kernel(dst_hbm, out_hbm, didx_v, hist_v):
    c = lax.axis_index("c")
    s = lax.axis_index("s")
    wid = s * NC + c
    pltpu.sync_copy(dst_hbm.at[pl.ds(wid * EPW, EPW)], didx_v)

    def zbody(i, carry):
        hist_v[pl.ds(i * L, L)] = jnp.zeros((L,), jnp.float32)
        return carry

    lax.fori_loop(0, NPAD // L, zbody, 0)

    ones = jnp.ones((L,), jnp.float32)

    def body(i, carry):
        idx = didx_v[pl.ds(i * L, L)]
        plsc.addupdate_scatter(hist_v, [idx], ones)
        return carry

    lax.fori_loop(0, EPW // L, body, 0)
    pltpu.sync_copy(hist_v, out_hbm.at[wid])


@functools.partial(
    pl.kernel,
    mesh=_MESH,
    out_type=jax.ShapeDtypeStruct((NC, NPAD, D), jnp.float32),
    scratch_types=[
        pltpu.VMEM((4, 2, CH), jnp.int32),
        pltpu.VMEM((CH, D), jnp.float32),
        pltpu.VMEM((CH, D), jnp.float32),
        pltpu.VMEM_SHARED((NPAD, D), jnp.float32),
        pltpu.SemaphoreType.DMA,
        pltpu.SemaphoreType.DMA,
        pltpu.SemaphoreType.DMA,
        pltpu.SemaphoreType.DMA,
    ],
)
def _scatter_kernel(hs_hbm, idx_hbm, out_hbm,
                    ibuf, rows0, rows1, acc_sh,
                    sem_g0, sem_g1, sem_i0, sem_i1):
    c = lax.axis_index("c")
    s = lax.axis_index("s")
    wid = s * NC + c
    rows = (rows0, rows1)
    gsems = (sem_g0, sem_g1)
    isems = (sem_i0, sem_i1)
    pltpu.sync_copy(hs_hbm.at[pl.ds(s * RPT, RPT)],
                    acc_sh.at[pl.ds(s * RPT, RPT)])
    plsc.subcore_barrier()

    def load_idx(ck, b):
        pltpu.async_copy(idx_hbm.at[wid, ck], ibuf.at[ck % 4], isems[b])

    def wait_idx(b):
        pltpu.make_async_copy(idx_hbm.at[wid, 0], ibuf.at[0], isems[b]).wait()

    def gather(ck, b):
        pltpu.async_copy(hs_hbm.at[ibuf.at[ck % 4, 0]], rows[b], gsems[b])

    def drain_gather(b):
        pltpu.make_async_copy(hs_hbm.at[pl.ds(0, CH)], rows[b],
                              gsems[b]).wait()

    def scatter(ck, b):
        pltpu.sync_copy(rows[b], acc_sh.at[ibuf.at[ck % 4, 1]], add=True)

    load_idx(0, 0)
    load_idx(1, 1)
    wait_idx(0)
    gather(0, 0)
    load_idx(2, 0)
    wait_idx(1)
    gather(1, 1)
    load_idx(3, 1)

    def body(i, carry):
        for b in range(2):
            ck = 2 * i + b
            drain_gather(b)
            scatter(ck, b)
            wait_idx(b)
            gather(ck + 2, b)
            load_idx(ck + 4, b)
        return carry

    lax.fori_loop(0, (NCHUNK - 5) // 2, body, 0)
    drain_gather(0)
    scatter(NCHUNK - 5, 0)
    wait_idx(0)
    gather(NCHUNK - 3, 0)
    load_idx(NCHUNK - 1, 0)
    drain_gather(1)
    scatter(NCHUNK - 4, 1)
    wait_idx(1)
    gather(NCHUNK - 2, 1)
    drain_gather(0)
    scatter(NCHUNK - 3, 0)
    wait_idx(0)
    gather(NCHUNK - 1, 0)
    drain_gather(1)
    scatter(NCHUNK - 2, 1)
    drain_gather(0)
    scatter(NCHUNK - 1, 0)

    plsc.subcore_barrier()
    pltpu.sync_copy(acc_sh.at[pl.ds(s * RPT, RPT)],
                    out_hbm.at[c, pl.ds(s * RPT, RPT)])


BLK = 1280


def _degsum_body(p_ref, cnt_ref):
    cnt_ref[...] = jnp.sum(p_ref[...], axis=0, keepdims=True)


def _degsum_call(degp):
    return pl.pallas_call(
        _degsum_body,
        out_shape=jax.ShapeDtypeStruct((1, NPAD), jnp.float32),
    )(degp)


def _l1_body(x_ref, deg_ref, w_ref, hs_ref, g_ref):
    g = lax.rsqrt(1.0 + deg_ref[...])
    h = jnp.dot(x_ref[...], w_ref[...], preferred_element_type=jnp.float32)
    hs_ref[...] = h * g
    g_ref[...] = g


def _l2_body(a0_ref, a1_ref, hs1_ref, g_ref, b_ref, w_ref, hs2_ref):
    g = g_ref[...]
    y = g * (a0_ref[...] + a1_ref[...] - hs1_ref[...]) + b_ref[...]
    y = jnp.maximum(y, 0.0)
    hs2_ref[...] = jnp.dot(y, w_ref[...],
                           preferred_element_type=jnp.float32) * g


def _l3_body(a0_ref, a1_ref, hs2_ref, g_ref, b_ref, out_ref):
    g = g_ref[...]
    out_ref[...] = g * (a0_ref[...] + a1_ref[...] - hs2_ref[...]) + b_ref[...]


_row = pl.BlockSpec((BLK, D), lambda i: (i, 0))
_gcol = pl.BlockSpec((BLK, 1), lambda i: (i, 0))
_full_w = pl.BlockSpec((D, D), lambda i: (0, 0))
_bias = pl.BlockSpec((1, D), lambda i: (0, 0))
_GRID = (NPAD // BLK,)


def _l1_call(x_pad, deg_col, W1):
    return pl.pallas_call(
        _l1_body,
        grid=_GRID,
        in_specs=[_row, _gcol, _full_w],
        out_specs=[_row, _gcol],
        out_shape=[jax.ShapeDtypeStruct((NPAD, D), jnp.float32),
                   jax.ShapeDtypeStruct((NPAD, 1), jnp.float32)],
    )(x_pad, deg_col, W1)


def _l2_call(a0, a1, hs1, g, b1, W2):
    return pl.pallas_call(
        _l2_body,
        grid=_GRID,
        in_specs=[_row, _row, _row, _gcol, _bias, _full_w],
        out_specs=_row,
        out_shape=jax.ShapeDtypeStruct((NPAD, D), jnp.float32),
    )(a0, a1, hs1, g, b1, W2)


def _l3_call(a0, a1, hs2, g, b2):
    return pl.pallas_call(
        _l3_body,
        grid=_GRID,
        in_specs=[_row, _row, _row, _gcol, _bias],
        out_specs=_row,
        out_shape=jax.ShapeDtypeStruct((NPAD, D), jnp.float32),
    )(a0, a1, hs2, g, b2)


def kernel(x, edge_index, W1, b1, W2, b2):
    src = edge_index[0].astype(jnp.int32)
    dst = edge_index[1].astype(jnp.int32)
    idx3 = jnp.stack([src.reshape(NW, NCHUNK, CH),
                      dst.reshape(NW, NCHUNK, CH)], axis=2)
    x_pad = jnp.zeros((NPAD, D), jnp.float32).at[:N].set(x)

    degp = _deg_kernel(dst)
    deg_col = _degsum_call(degp).reshape(NPAD, 1)
    hs1, g = _l1_call(x_pad, deg_col, W1)
    acc1 = _scatter_kernel(hs1, idx3)
    hs2 = _l2_call(acc1[0], acc1[1], hs1, g,
                   b1.reshape(1, D).astype(jnp.float32), W2)
    acc2 = _scatter_kernel(hs2, idx3)
    out = _l3_call(acc2[0], acc2[1], hs2, g,
                   b2.reshape(1, D).astype(jnp.float32))
    return out[:N]


# speedup vs baseline: 29.5108x; 1.0294x over previous
import functools

import jax
import jax.numpy as jnp
from jax import lax
from jax.experimental import pallas as pl
from jax.experimental.pallas import tpu as pltpu
from jax.experimental.pallas import tpu_sc as plsc

N = 10000
E = 320000
D = 128
NC, NS, L = 2, 16, 16
NW = NC * NS
NPAD = 10240
EPW = E // NW
CH = 80
NCHUNK = EPW // CH
RPT = NPAD // NS

_MESH = plsc.VectorSubcoreMesh(core_axis_name="c", subcore_axis_name="s")


@functools.partial(
    pl.kernel,
    mesh=_MESH,
    compiler_params=pltpu.CompilerParams(needs_layout_passes=False),
    out_type=jax.ShapeDtypeStruct((NW, NPAD), jnp.float32),
    scratch_types=[
        pltpu.VMEM((EPW,), jnp.int32),
        pltpu.VMEM((NPAD,), jnp.float32),
    ],
)
def _deg_kernel(dst_hbm, out_hbm, didx_v, hist_v):
    c = lax.axis_index("c")
    s = lax.axis_index("s")
    wid = s * NC + c
    pltpu.sync_copy(dst_hbm.at[pl.ds(wid * EPW, EPW)], didx_v)

    def zbody(i, carry):
        hist_v[pl.ds(i * L, L)] = jnp.zeros((L,), jnp.float32)
        return carry

    lax.fori_loop(0, NPAD // L, zbody, 0)

    ones = jnp.ones((L,), jnp.float32)

    def body(i, carry):
        idx = didx_v[pl.ds(i * L, L)]
        plsc.addupdate_scatter(hist_v, [idx], ones)
        return carry

    lax.fori_loop(0, EPW // L, body, 0)
    pltpu.sync_copy(hist_v, out_hbm.at[wid])


@functools.partial(
    pl.kernel,
    mesh=_MESH,
    out_type=jax.ShapeDtypeStruct((NC, NPAD, D), jnp.float32),
    scratch_types=[
        pltpu.VMEM((4, 2, CH), jnp.int32),
        pltpu.VMEM((CH, D), jnp.float32),
        pltpu.VMEM((CH, D), jnp.float32),
        pltpu.VMEM_SHARED((NPAD, D), jnp.float32),
        pltpu.SemaphoreType.DMA,
        pltpu.SemaphoreType.DMA,
        pltpu.SemaphoreType.DMA,
        pltpu.SemaphoreType.DMA,
    ],
)
def _scatter_kernel(hs_hbm, idx_hbm, out_hbm,
                    ibuf, rows0, rows1, acc_sh,
                    sem_g0, sem_g1, sem_i0, sem_i1):
    c = lax.axis_index("c")
    s = lax.axis_index("s")
    wid = s * NC + c
    rows = (rows0, rows1)
    gsems = (sem_g0, sem_g1)
    isems = (sem_i0, sem_i1)
    pltpu.sync_copy(hs_hbm.at[pl.ds(s * RPT, RPT)],
                    acc_sh.at[pl.ds(s * RPT, RPT)])
    plsc.subcore_barrier()

    def load_idx(ck, b):
        pltpu.async_copy(idx_hbm.at[wid, ck], ibuf.at[ck % 4], isems[b])

    def wait_idx(b):
        pltpu.make_async_copy(idx_hbm.at[wid, 0], ibuf.at[0], isems[b]).wait()

    def gather(ck, b):
        pltpu.async_copy(hs_hbm.at[ibuf.at[ck % 4, 0]], rows[b], gsems[b])

    def drain_gather(b):
        pltpu.make_async_copy(hs_hbm.at[pl.ds(0, CH)], rows[b],
                              gsems[b]).wait()

    def scatter(ck, b):
        pltpu.sync_copy(rows[b], acc_sh.at[ibuf.at[ck % 4, 1]], add=True)

    load_idx(0, 0)
    load_idx(1, 1)
    wait_idx(0)
    gather(0, 0)
    load_idx(2, 0)
    wait_idx(1)
    gather(1, 1)
    load_idx(3, 1)

    def body(i, carry):
        for b in range(2):
            ck = 2 * i + b
            drain_gather(b)
            scatter(ck, b)
            wait_idx(b)
            gather(ck + 2, b)
            load_idx(ck + 4, b)
        return carry

    lax.fori_loop(0, (NCHUNK - 5) // 2, body, 0)
    drain_gather(0)
    scatter(NCHUNK - 5, 0)
    wait_idx(0)
    gather(NCHUNK - 3, 0)
    load_idx(NCHUNK - 1, 0)
    drain_gather(1)
    scatter(NCHUNK - 4, 1)
    wait_idx(1)
    gather(NCHUNK - 2, 1)
    drain_gather(0)
    scatter(NCHUNK - 3, 0)
    wait_idx(0)
    gather(NCHUNK - 1, 0)
    drain_gather(1)
    scatter(NCHUNK - 2, 1)
    drain_gather(0)
    scatter(NCHUNK - 1, 0)

    plsc.subcore_barrier()
    pltpu.sync_copy(acc_sh.at[pl.ds(s * RPT, RPT)],
                    out_hbm.at[c, pl.ds(s * RPT, RPT)])


BLK = 1280


def _l1_body(x_ref, degp_ref, w_ref, hs_ref, g_ref):
    deg = jnp.sum(degp_ref[...], axis=0)
    g = lax.rsqrt(1.0 + deg)[:, None]
    h = jnp.dot(x_ref[...], w_ref[...], preferred_element_type=jnp.float32)
    hs_ref[...] = h * g
    g_ref[...] = g


def _l2_body(a0_ref, a1_ref, hs1_ref, g_ref, b_ref, w_ref, hs2_ref):
    g = g_ref[...]
    y = g * (a0_ref[...] + a1_ref[...] - hs1_ref[...]) + b_ref[...]
    y = jnp.maximum(y, 0.0)
    hs2_ref[...] = jnp.dot(y, w_ref[...],
                           preferred_element_type=jnp.float32) * g


def _l3_body(a0_ref, a1_ref, hs2_ref, g_ref, b_ref, out_ref):
    g = g_ref[...]
    out_ref[...] = g * (a0_ref[...] + a1_ref[...] - hs2_ref[...]) + b_ref[...]


_row = pl.BlockSpec((BLK, D), lambda i: (i, 0))
_gcol = pl.BlockSpec((BLK, 1), lambda i: (i, 0))
_degp = pl.BlockSpec((NW, BLK), lambda i: (0, i))
_full_w = pl.BlockSpec((D, D), lambda i: (0, 0))
_bias = pl.BlockSpec((1, D), lambda i: (0, 0))
_GRID = (NPAD // BLK,)


def _l1_call(x_pad, degp, W1):
    return pl.pallas_call(
        _l1_body,
        grid=_GRID,
        in_specs=[_row, _degp, _full_w],
        out_specs=[_row, _gcol],
        out_shape=[jax.ShapeDtypeStruct((NPAD, D), jnp.float32),
                   jax.ShapeDtypeStruct((NPAD, 1), jnp.float32)],
    )(x_pad, degp, W1)


def _l2_call(a0, a1, hs1, g, b1, W2):
    return pl.pallas_call(
        _l2_body,
        grid=_GRID,
        in_specs=[_row, _row, _row, _gcol, _bias, _full_w],
        out_specs=_row,
        out_shape=jax.ShapeDtypeStruct((NPAD, D), jnp.float32),
    )(a0, a1, hs1, g, b1, W2)


def _l3_call(a0, a1, hs2, g, b2):
    return pl.pallas_call(
        _l3_body,
        grid=_GRID,
        in_specs=[_row, _row, _row, _gcol, _bias],
        out_specs=_row,
        out_shape=jax.ShapeDtypeStruct((N, D), jnp.float32),
    )(a0, a1, hs2, g, b2)


def kernel(x, edge_index, W1, b1, W2, b2):
    src = edge_index[0].astype(jnp.int32)
    dst = edge_index[1].astype(jnp.int32)
    idx3 = jnp.stack([src.reshape(NW, NCHUNK, CH),
                      dst.reshape(NW, NCHUNK, CH)], axis=2)
    x_pad = jnp.zeros((NPAD, D), jnp.float32).at[:N].set(x)

    degp = _deg_kernel(dst)
    hs1, g = _l1_call(x_pad, degp, W1)
    acc1 = _scatter_kernel(hs1, idx3)
    hs2 = _l2_call(acc1[0], acc1[1], hs1, g,
                   b1.reshape(1, D).astype(jnp.float32), W2)
    acc2 = _scatter_kernel(hs2, idx3)
    return _l3_call(acc2[0], acc2[1], hs2, g,
                    b2.reshape(1, D).astype(jnp.float32))


# speedup vs baseline: 33.8920x; 1.1485x over previous
import functools

import jax
import jax.numpy as jnp
from jax import lax
from jax.experimental import pallas as pl
from jax.experimental.pallas import tpu as pltpu
from jax.experimental.pallas import tpu_sc as plsc

N = 10000
E = 320000
D = 128
NC, NS, L = 2, 16, 16
NW = NC * NS
NPAD = 10240
EPW = E // NW
CH = 80
NCHUNK = EPW // CH
RPT = NPAD // NS

_MESH = plsc.VectorSubcoreMesh(core_axis_name="c", subcore_axis_name="s")


@functools.partial(
    pl.kernel,
    mesh=_MESH,
    compiler_params=pltpu.CompilerParams(needs_layout_passes=False),
    out_type=jax.ShapeDtypeStruct((NW, NPAD), jnp.float32),
    scratch_types=[
        pltpu.VMEM((EPW,), jnp.int32),
        pltpu.VMEM((NPAD,), jnp.float32),
    ],
)
def _deg_kernel(dst_hbm, out_hbm, didx_v, hist_v):
    c = lax.axis_index("c")
    s = lax.axis_index("s")
    wid = s * NC + c
    pltpu.sync_copy(dst_hbm.at[pl.ds(wid * EPW, EPW)], didx_v)

    def zbody(i, carry):
        hist_v[pl.ds(i * L, L)] = jnp.zeros((L,), jnp.float32)
        return carry

    lax.fori_loop(0, NPAD // L, zbody, 0)

    ones = jnp.ones((L,), jnp.float32)

    def body(i, carry):
        idx = didx_v[pl.ds(i * L, L)]
        plsc.addupdate_scatter(hist_v, [idx], ones)
        return carry

    lax.fori_loop(0, EPW // L, body, 0)
    pltpu.sync_copy(hist_v, out_hbm.at[wid])


@functools.partial(
    pl.kernel,
    mesh=_MESH,
    out_type=jax.ShapeDtypeStruct((NC, NPAD, D), jnp.float32),
    scratch_types=[
        pltpu.VMEM((6, 2, CH), jnp.int32),
        pltpu.VMEM((CH, D), jnp.float32),
        pltpu.VMEM((CH, D), jnp.float32),
        pltpu.VMEM((CH, D), jnp.float32),
        pltpu.VMEM_SHARED((NPAD, D), jnp.float32),
        pltpu.SemaphoreType.DMA,
        pltpu.SemaphoreType.DMA,
        pltpu.SemaphoreType.DMA,
        pltpu.SemaphoreType.DMA,
        pltpu.SemaphoreType.DMA,
        pltpu.SemaphoreType.DMA,
    ],
)
def _scatter_kernel(hs_hbm, idx_hbm, out_hbm,
                    ibuf, rows0, rows1, rows2, acc_sh,
                    sem_g0, sem_g1, sem_g2, sem_i0, sem_i1, sem_i2):
    c = lax.axis_index("c")
    s = lax.axis_index("s")
    wid = s * NC + c
    rows = (rows0, rows1, rows2)
    gsems = (sem_g0, sem_g1, sem_g2)
    isems = (sem_i0, sem_i1, sem_i2)
    pltpu.sync_copy(hs_hbm.at[pl.ds(s * RPT, RPT)],
                    acc_sh.at[pl.ds(s * RPT, RPT)])
    plsc.subcore_barrier()

    def load_idx(ck, sb):
        pltpu.async_copy(idx_hbm.at[wid, ck], ibuf.at[ck % 6], isems[sb])

    def wait_idx(sb):
        pltpu.make_async_copy(idx_hbm.at[wid, 0], ibuf.at[0], isems[sb]).wait()

    def gather(ck, rb):
        pltpu.async_copy(hs_hbm.at[ibuf.at[ck % 6, 0]], rows[rb], gsems[rb])

    def drain_gather(rb):
        pltpu.make_async_copy(hs_hbm.at[pl.ds(0, CH)], rows[rb],
                              gsems[rb]).wait()

    def scatter(ck, rb):
        pltpu.sync_copy(rows[rb], acc_sh.at[ibuf.at[ck % 6, 1]], add=True)

    load_idx(0, 0)
    load_idx(1, 1)
    load_idx(2, 2)
    wait_idx(0)
    gather(0, 0)
    load_idx(3, 0)
    wait_idx(1)
    gather(1, 1)
    load_idx(4, 1)
    wait_idx(2)
    gather(2, 2)

    def body(i, carry):
        for b in range(3):
            ck = 3 * i + b
            drain_gather(b)
            scatter(ck, b)
            wait_idx(b)
            gather(ck + 3, b)
            load_idx(ck + 5, (b + 2) % 3)
        return carry

    lax.fori_loop(0, (NCHUNK - 5) // 3, body, 0)
    drain_gather(0)
    scatter(NCHUNK - 5, 0)
    wait_idx(0)
    gather(NCHUNK - 2, 0)
    drain_gather(1)
    scatter(NCHUNK - 4, 1)
    wait_idx(1)
    gather(NCHUNK - 1, 1)
    drain_gather(2)
    scatter(NCHUNK - 3, 2)
    drain_gather(0)
    scatter(NCHUNK - 2, 0)
    drain_gather(1)
    scatter(NCHUNK - 1, 1)

    plsc.subcore_barrier()
    pltpu.sync_copy(acc_sh.at[pl.ds(s * RPT, RPT)],
                    out_hbm.at[c, pl.ds(s * RPT, RPT)])


BLK = 1280


def _l1_body(x_ref, degp_ref, w_ref, hs_ref, g_ref):
    deg = jnp.sum(degp_ref[...], axis=0)
    g = lax.rsqrt(1.0 + deg)[:, None]
    h = jnp.dot(x_ref[...], w_ref[...], preferred_element_type=jnp.float32)
    hs_ref[...] = h * g
    g_ref[...] = g


def _l2_body(a0_ref, a1_ref, hs1_ref, g_ref, b_ref, w_ref, hs2_ref):
    g = g_ref[...]
    y = g * (a0_ref[...] + a1_ref[...] - hs1_ref[...]) + b_ref[...]
    y = jnp.maximum(y, 0.0)
    hs2_ref[...] = jnp.dot(y, w_ref[...],
                           preferred_element_type=jnp.float32) * g


def _l3_body(a0_ref, a1_ref, hs2_ref, g_ref, b_ref, out_ref):
    g = g_ref[...]
    out_ref[...] = g * (a0_ref[...] + a1_ref[...] - hs2_ref[...]) + b_ref[...]


_row = pl.BlockSpec((BLK, D), lambda i: (i, 0))
_gcol = pl.BlockSpec((BLK, 1), lambda i: (i, 0))
_degp = pl.BlockSpec((NW, BLK), lambda i: (0, i))
_full_w = pl.BlockSpec((D, D), lambda i: (0, 0))
_bias = pl.BlockSpec((1, D), lambda i: (0, 0))
_GRID = (NPAD // BLK,)


def _l1_call(x_pad, degp, W1):
    return pl.pallas_call(
        _l1_body,
        grid=_GRID,
        in_specs=[_row, _degp, _full_w],
        out_specs=[_row, _gcol],
        out_shape=[jax.ShapeDtypeStruct((NPAD, D), jnp.float32),
                   jax.ShapeDtypeStruct((NPAD, 1), jnp.float32)],
    )(x_pad, degp, W1)


def _l2_call(a0, a1, hs1, g, b1, W2):
    return pl.pallas_call(
        _l2_body,
        grid=_GRID,
        in_specs=[_row, _row, _row, _gcol, _bias, _full_w],
        out_specs=_row,
        out_shape=jax.ShapeDtypeStruct((NPAD, D), jnp.float32),
    )(a0, a1, hs1, g, b1, W2)


def _l3_call(a0, a1, hs2, g, b2):
    return pl.pallas_call(
        _l3_body,
        grid=_GRID,
        in_specs=[_row, _row, _row, _gcol, _bias],
        out_specs=_row,
        out_shape=jax.ShapeDtypeStruct((N, D), jnp.float32),
    )(a0, a1, hs2, g, b2)


def kernel(x, edge_index, W1, b1, W2, b2):
    src = edge_index[0].astype(jnp.int32)
    dst = edge_index[1].astype(jnp.int32)
    idx3 = jnp.stack([src.reshape(NW, NCHUNK, CH),
                      dst.reshape(NW, NCHUNK, CH)], axis=2)
    x_pad = jnp.zeros((NPAD, D), jnp.float32).at[:N].set(x)

    degp = _deg_kernel(dst)
    hs1, g = _l1_call(x_pad, degp, W1)
    acc1 = _scatter_kernel(hs1, idx3)
    hs2 = _l2_call(acc1[0], acc1[1], hs1, g,
                   b1.reshape(1, D).astype(jnp.float32), W2)
    acc2 = _scatter_kernel(hs2, idx3)
    return _l3_call(acc2[0], acc2[1], hs2, g,
                    b2.reshape(1, D).astype(jnp.float32))
